# baseline (speedup 1.0000x reference)
import functools

import jax
import jax.numpy as jnp
from jax import lax
from jax.experimental import pallas as pl
from jax.experimental.pallas import tpu as pltpu
from jax.experimental.pallas import tpu_sc as plsc

NC, NS, L = 2, 16, 16
NW = NC * NS
D = 768
ROWS = 64 * 576
PER_W = ROWS // NW
CROWS = 16
N_CHUNKS = PER_W // CROWS
NBUF = 4


@functools.partial(
    pl.kernel,
    out_type=jax.ShapeDtypeStruct((ROWS, D), jnp.float32),
    mesh=plsc.VectorSubcoreMesh(core_axis_name="c", subcore_axis_name="s"),
    scratch_types=[
        pltpu.VMEM((CROWS, D), jnp.float32),
        pltpu.VMEM((CROWS, D), jnp.float32),
        pltpu.VMEM((CROWS, D), jnp.float32),
        pltpu.VMEM((CROWS, D), jnp.float32),
        pltpu.VMEM((CROWS, D), jnp.float32),
        pltpu.VMEM((CROWS, D), jnp.float32),
        pltpu.VMEM((CROWS, D), jnp.float32),
        pltpu.VMEM((CROWS, D), jnp.float32),
        pltpu.VMEM((8, L), jnp.float32),
        pltpu.SemaphoreType.DMA,
        pltpu.SemaphoreType.DMA,
        pltpu.SemaphoreType.DMA,
        pltpu.SemaphoreType.DMA,
        pltpu.SemaphoreType.DMA,
        pltpu.SemaphoreType.DMA,
        pltpu.SemaphoreType.DMA,
        pltpu.SemaphoreType.DMA,
    ],
)
def _sc_quantize(x_hbm, params_hbm, out_hbm, i0, i1, i2, i3, o0, o1, o2, o3,
                 pbuf, si0, si1, si2, si3, so0, so1, so2, so3):
    ibufs = (i0, i1, i2, i3)
    obufs = (o0, o1, o2, o3)
    sin = (si0, si1, si2, si3)
    sout = (so0, so1, so2, so3)
    wid = lax.axis_index("s") * NC + lax.axis_index("c")
    base = wid * PER_W
    pltpu.sync_copy(params_hbm, pbuf)
    c0 = pbuf[0]
    c1 = pbuf[1]
    c2 = pbuf[2]
    t01 = pbuf[3]
    t12 = pbuf[4]

    for b in range(NBUF):
        pltpu.async_copy(
            x_hbm.at[pl.ds(base + b * CROWS, CROWS)], ibufs[b], sin[b])

    @pl.loop(0, N_CHUNKS, step=NBUF)
    def _group(g0):
        for b in range(NBUF):
            row0 = base + (g0 + b) * CROWS
            pltpu.make_async_copy(
                x_hbm.at[pl.ds(row0, CROWS)], ibufs[b], sin[b]).wait()

            @pl.when(g0 > 0)
            def _drain(b=b, row0=row0):
                pltpu.make_async_copy(
                    obufs[b], out_hbm.at[pl.ds(row0 - NBUF * CROWS, CROWS)],
                    sout[b]).wait()

            ib = ibufs[b]
            ob = obufs[b]

            @plsc.parallel_loop(0, CROWS, step=1, unroll=8)
            def _row(r, ib=ib, ob=ob):
                for c in range(0, D, L):
                    xv = ib[r, pl.ds(c, L)]
                    ob[r, pl.ds(c, L)] = jnp.where(
                        xv <= t01, c0, jnp.where(xv <= t12, c1, c2))

            pltpu.async_copy(ob, out_hbm.at[pl.ds(row0, CROWS)], sout[b])

            @pl.when(row0 + NBUF * CROWS < base + PER_W)
            def _prefetch(b=b, row0=row0):
                pltpu.async_copy(
                    x_hbm.at[pl.ds(row0 + NBUF * CROWS, CROWS)],
                    ibufs[b], sin[b])

    last0 = base + (N_CHUNKS - NBUF) * CROWS
    for b in range(NBUF):
        pltpu.make_async_copy(
            obufs[b], out_hbm.at[pl.ds(last0 + b * CROWS, CROWS)],
            sout[b]).wait()


def kernel(x, u, centers, temperature):
    del u, temperature
    c0 = centers[0]
    c1 = centers[1]
    c2 = centers[2]
    rows = jnp.stack([c0, c1, c2, (c0 + c1) * 0.5, (c1 + c2) * 0.5,
                      jnp.float32(0), jnp.float32(0), jnp.float32(0)])
    params = jnp.broadcast_to(rows[:, None], (8, L)).astype(jnp.float32)
    out = _sc_quantize(x.reshape(ROWS, D), params)
    return out.reshape(x.shape)


# speedup vs baseline: 1.4662x; 1.4662x over previous
import functools

import jax
import jax.numpy as jnp
from jax import lax
from jax.experimental import pallas as pl
from jax.experimental.pallas import tpu as pltpu
from jax.experimental.pallas import tpu_sc as plsc

NC, NS, L = 2, 16, 16
NW = NC * NS
D = 768
ROWS = 64 * 576
PER_W = ROWS // NW
CROWS = 16
N_CHUNKS = PER_W // CROWS
NBUF = 4


@functools.partial(
    pl.kernel,
    out_type=jax.ShapeDtypeStruct((ROWS, D), jnp.float32),
    mesh=plsc.VectorSubcoreMesh(core_axis_name="c", subcore_axis_name="s"),
    scratch_types=[
        pltpu.VMEM((CROWS, D), jnp.float32),
        pltpu.VMEM((CROWS, D), jnp.float32),
        pltpu.VMEM((CROWS, D), jnp.float32),
        pltpu.VMEM((CROWS, D), jnp.float32),
        pltpu.VMEM((CROWS, D), jnp.float32),
        pltpu.VMEM((CROWS, D), jnp.float32),
        pltpu.VMEM((CROWS, D), jnp.float32),
        pltpu.VMEM((CROWS, D), jnp.float32),
        pltpu.VMEM((8, L), jnp.float32),
        pltpu.SemaphoreType.DMA,
        pltpu.SemaphoreType.DMA,
        pltpu.SemaphoreType.DMA,
        pltpu.SemaphoreType.DMA,
        pltpu.SemaphoreType.DMA,
        pltpu.SemaphoreType.DMA,
        pltpu.SemaphoreType.DMA,
        pltpu.SemaphoreType.DMA,
    ],
)
def _sc_quantize(x_hbm, params_hbm, out_hbm, i0, i1, i2, i3, o0, o1, o2, o3,
                 pbuf, si0, si1, si2, si3, so0, so1, so2, so3):
    ibufs = (i0, i1, i2, i3)
    obufs = (o0, o1, o2, o3)
    sin = (si0, si1, si2, si3)
    sout = (so0, so1, so2, so3)
    wid = lax.axis_index("s") * NC + lax.axis_index("c")
    base = wid * PER_W
    pltpu.sync_copy(params_hbm, pbuf)
    c0 = pbuf[0]
    c1 = pbuf[1]
    c2 = pbuf[2]
    t01 = pbuf[3]
    t12 = pbuf[4]

    for b in range(NBUF):
        pltpu.async_copy(
            x_hbm.at[pl.ds(base + b * CROWS, CROWS)], ibufs[b], sin[b])

    @pl.loop(0, N_CHUNKS, step=NBUF)
    def _group(g0):
        for b in range(NBUF):
            row0 = base + (g0 + b) * CROWS
            pltpu.make_async_copy(
                x_hbm.at[pl.ds(row0, CROWS)], ibufs[b], sin[b]).wait()

            @pl.when(g0 > 0)
            def _drain(b=b, row0=row0):
                pltpu.make_async_copy(
                    obufs[b], out_hbm.at[pl.ds(row0 - NBUF * CROWS, CROWS)],
                    sout[b]).wait()

            ib = ibufs[b]
            ob = obufs[b]

            @plsc.parallel_loop(0, CROWS, step=1, unroll=4)
            def _row(r, ib=ib, ob=ob):
                for c in range(0, D, L):
                    xv = ib[r, pl.ds(c, L)]
                    ob[r, pl.ds(c, L)] = jnp.where(
                        xv <= t01, c0, jnp.where(xv <= t12, c1, c2))

            pltpu.async_copy(ob, out_hbm.at[pl.ds(row0, CROWS)], sout[b])

            @pl.when(row0 + NBUF * CROWS < base + PER_W)
            def _prefetch(b=b, row0=row0):
                pltpu.async_copy(
                    x_hbm.at[pl.ds(row0 + NBUF * CROWS, CROWS)],
                    ibufs[b], sin[b])

    last0 = base + (N_CHUNKS - NBUF) * CROWS
    for b in range(NBUF):
        pltpu.make_async_copy(
            obufs[b], out_hbm.at[pl.ds(last0 + b * CROWS, CROWS)],
            sout[b]).wait()


def kernel(x, u, centers, temperature):
    del u, temperature
    c0 = centers[0]
    c1 = centers[1]
    c2 = centers[2]
    rows = jnp.stack([c0, c1, c2, (c0 + c1) * 0.5, (c1 + c2) * 0.5,
                      jnp.float32(0), jnp.float32(0), jnp.float32(0)])
    params = jnp.broadcast_to(rows[:, None], (8, L)).astype(jnp.float32)
    out = _sc_quantize(x.reshape(ROWS, D), params)
    return out.reshape(x.shape)
